# baseline (speedup 1.0000x reference)
import functools

import jax
import jax.numpy as jnp
from jax import lax
from jax.experimental import pallas as pl
from jax.experimental.pallas import tpu as pltpu
from jax.experimental.pallas import tpu_sc as plsc

_L = 16


def _sc_embed(x_flat, table_flat, rep_table, D):
    (P,) = x_flat.shape
    info = plsc.get_sparse_core_info()
    NC, NS = info.num_cores, info.num_subcores
    NW = NC * NS
    per_w = P // NW
    K = 256
    KC = 160
    KG = K - KC
    NB = 2
    n_outer = per_w // (K * NB)
    n_sub = D // _L
    nqg = KG // _L

    mesh = plsc.VectorSubcoreMesh(core_axis_name="c", subcore_axis_name="s")

    @functools.partial(
        pl.kernel,
        mesh=mesh,
        out_type=jax.ShapeDtypeStruct((P, D), jnp.float32),
        scratch_types=[
            pltpu.VMEM((per_w,), jnp.float32),
            pltpu.VMEM((2 * D,), jnp.float32),
            pltpu.VMEM((NB, KC, D), jnp.float32),
            pltpu.VMEM((NB, KG, D), jnp.float32),
            pltpu.VMEM((NB, KG), jnp.int32),
            pltpu.SemaphoreType.DMA,
            pltpu.SemaphoreType.DMA,
            pltpu.SemaphoreType.DMA,
            pltpu.SemaphoreType.DMA,
        ],
    )
    def body(x_hbm, tbl_hbm, rep_hbm, out_hbm, xv, tblv, rowsc, rowsg, idxg,
             semo0, semo1, semg0, semg1):
        wid = lax.axis_index("s") * NC + lax.axis_index("c")
        base = wid * per_w
        pltpu.sync_copy(x_hbm.at[pl.ds(base, per_w)], xv)
        pltpu.sync_copy(tbl_hbm.at[pl.ds(0, 2 * D)], tblv)
        semo = (semo0, semo1)
        semg = (semg0, semg1)

        t0 = [tblv[pl.ds(k * _L, _L)] for k in range(n_sub)]
        td = [tblv[pl.ds(D + k * _L, _L)] - t0[k] for k in range(n_sub)]
        ones = jnp.full((_L,), 1.0, jnp.float32)
        zeros = jnp.full((_L,), 0.0, jnp.float32)
        lanes = jnp.arange(_L, dtype=jnp.int32)
        gpat = [2 * (lanes + q * _L) for q in range(nqg)]
        onei = jnp.full((_L,), 1, jnp.int32)
        zeroi = jnp.full((_L,), 0, jnp.int32)

        def outer(c, carry):
            for b in range(NB):
                rows_b = rowsc.at[b]
                pos0 = c * (K * NB) + b * K

                @pl.when(c > 0)
                def _wait():
                    pltpu.make_async_copy(
                        rowsc.at[b], out_hbm.at[pl.ds(0, KC)],
                        semo[b],
                    ).wait()
                    pltpu.make_async_copy(
                        rowsg.at[b], out_hbm.at[pl.ds(0, KG)],
                        semo[b],
                    ).wait()

                for q in range(nqg):
                    xq = xv[pl.ds(pos0 + KC + q * _L, _L)]
                    s = jnp.where(xq > 0.0, onei, zeroi)
                    idxg.at[b][pl.ds(q * _L, _L)] = gpat[q] + s
                pltpu.async_copy(rep_hbm.at[idxg.at[b]], rowsg.at[b], semg[b])

                def pos16(ii, carry2):
                    xvec = xv[pl.ds(pos0 + ii * _L, _L)]
                    svec = jnp.where(xvec > 0.0, ones, zeros)
                    for j in range(_L):
                        sj = jnp.broadcast_to(svec[j], (_L,))
                        o = ii * _L + j
                        for k in range(n_sub):
                            rows_b[o, pl.ds(k * _L, _L)] = t0[k] + sj * td[k]
                    return carry2

                lax.fori_loop(0, KC // _L, pos16, 0)

                pltpu.make_async_copy(
                    rep_hbm.at[idxg.at[b]], rowsg.at[b], semg[b]
                ).wait()
                pltpu.async_copy(
                    rows_b, out_hbm.at[pl.ds(base + pos0, KC)],
                    semo[b],
                )
                pltpu.async_copy(
                    rowsg.at[b],
                    out_hbm.at[pl.ds(base + pos0 + KC, KG)],
                    semo[b],
                )
            return carry

        lax.fori_loop(0, n_outer, outer, 0)
        for b in range(NB):
            pltpu.make_async_copy(
                rowsc.at[b], out_hbm.at[pl.ds(0, KC)], semo[b]
            ).wait()
            pltpu.make_async_copy(
                rowsg.at[b], out_hbm.at[pl.ds(0, KG)], semo[b]
            ).wait()

    return body(x_flat, table_flat, rep_table)


def kernel(x, embedding):
    B, N = x.shape
    V, D = embedding.shape
    rep = jnp.tile(embedding[:2], (96, 1))
    out = _sc_embed(x.reshape(B * N), embedding.reshape(V * D), rep, D)
    return out.reshape(B, N, D)


# speedup vs baseline: 1.7460x; 1.7460x over previous
import functools

import jax
import jax.numpy as jnp
from jax import lax
from jax.experimental import pallas as pl
from jax.experimental.pallas import tpu as pltpu
from jax.experimental.pallas import tpu_sc as plsc

_L = 16


def _sc_embed(x_flat, table_flat, D):
    (P,) = x_flat.shape
    info = plsc.get_sparse_core_info()
    NC, NS = info.num_cores, info.num_subcores
    NW = NC * NS
    per_w = P // NW
    K = 256
    NB = 2
    n_outer = per_w // (K * NB)
    KD = K * D
    n_sub = D // _L

    mesh = plsc.VectorSubcoreMesh(core_axis_name="c", subcore_axis_name="s")

    @functools.partial(
        pl.kernel,
        mesh=mesh,
        out_type=jax.ShapeDtypeStruct((P * D,), jnp.float32),
        scratch_types=[
            pltpu.VMEM((per_w,), jnp.float32),
            pltpu.VMEM((2 * D,), jnp.float32),
            pltpu.VMEM((NB, KD), jnp.float32),
            pltpu.SemaphoreType.DMA,
            pltpu.SemaphoreType.DMA,
        ],
    )
    def body(x_hbm, tbl_hbm, out_hbm, xv, tblv, rows, sem0, sem1):
        wid = lax.axis_index("s") * NC + lax.axis_index("c")
        base = wid * per_w
        pltpu.sync_copy(x_hbm.at[pl.ds(base, per_w)], xv)
        pltpu.sync_copy(tbl_hbm.at[pl.ds(0, 2 * D)], tblv)
        sems = (sem0, sem1)

        t0 = [tblv[pl.ds(k * _L, _L)] for k in range(n_sub)]
        td = [tblv[pl.ds(D + k * _L, _L)] - t0[k] for k in range(n_sub)]
        ones = jnp.full((_L,), 1.0, jnp.float32)
        zeros = jnp.full((_L,), 0.0, jnp.float32)

        def outer(c, carry):
            for b in range(NB):
                rows_b = rows.at[b]
                pos0 = c * (K * NB) + b * K

                @pl.when(c > 0)
                def _wait():
                    pltpu.make_async_copy(
                        rows_b, out_hbm.at[pl.ds(base * D, KD)], sems[b]
                    ).wait()

                def pos16(ii):
                    xvec = xv[pl.ds(pos0 + ii * _L, _L)]
                    svec = jnp.where(xvec > 0.0, ones, zeros)
                    for j in range(_L):
                        sj = jnp.broadcast_to(svec[j], (_L,))
                        o = (ii * _L + j) * D
                        for k in range(n_sub):
                            rows_b[pl.ds(o + k * _L, _L)] = t0[k] + sj * td[k]

                plsc.parallel_loop(0, K // _L, 1, unroll=2)(pos16)
                pltpu.async_copy(
                    rows_b, out_hbm.at[pl.ds((base + pos0) * D, KD)], sems[b]
                )
            return carry

        lax.fori_loop(0, n_outer, outer, 0)
        for b in range(NB):
            pltpu.make_async_copy(
                rows.at[b], out_hbm.at[pl.ds(base * D, KD)], sems[b]
            ).wait()

    return body(x_flat, table_flat)


def kernel(x, embedding):
    B, N = x.shape
    V, D = embedding.shape
    out = _sc_embed(x.reshape(B * N), embedding.reshape(V * D), D)
    return out.reshape(B, N, D)


# speedup vs baseline: 1.7579x; 1.0068x over previous
import functools

import jax
import jax.numpy as jnp
from jax import lax
from jax.experimental import pallas as pl
from jax.experimental.pallas import tpu as pltpu
from jax.experimental.pallas import tpu_sc as plsc

_L = 16


def _sc_embed(x_flat, table_flat, D):
    (P,) = x_flat.shape
    info = plsc.get_sparse_core_info()
    NC, NS = info.num_cores, info.num_subcores
    NW = NC * NS
    per_w = P // NW
    K = 256
    NB = 2
    n_outer = per_w // (K * NB)
    KD = K * D
    n_sub = D // _L

    mesh = plsc.VectorSubcoreMesh(core_axis_name="c", subcore_axis_name="s")

    @functools.partial(
        pl.kernel,
        mesh=mesh,
        out_type=jax.ShapeDtypeStruct((P * D,), jnp.float32),
        scratch_types=[
            pltpu.VMEM((per_w,), jnp.float32),
            pltpu.VMEM((2 * D,), jnp.float32),
            pltpu.VMEM((NB, KD), jnp.float32),
            pltpu.SemaphoreType.DMA,
            pltpu.SemaphoreType.DMA,
        ],
    )
    def body(x_hbm, tbl_hbm, out_hbm, xv, tblv, rows, sem0, sem1):
        wid = lax.axis_index("s") * NC + lax.axis_index("c")
        base = wid * per_w
        pltpu.sync_copy(x_hbm.at[pl.ds(base, per_w)], xv)
        pltpu.sync_copy(tbl_hbm.at[pl.ds(0, 2 * D)], tblv)
        sems = (sem0, sem1)

        t0 = [tblv[pl.ds(k * _L, _L)] for k in range(n_sub)]
        td = [tblv[pl.ds(D + k * _L, _L)] - t0[k] for k in range(n_sub)]
        ones = jnp.full((_L,), 1.0, jnp.float32)
        zeros = jnp.full((_L,), 0.0, jnp.float32)

        def outer(c, carry):
            for b in range(NB):
                rows_b = rows.at[b]
                pos0 = c * (K * NB) + b * K

                @pl.when(c > 0)
                def _wait():
                    pltpu.make_async_copy(
                        rows_b, out_hbm.at[pl.ds(base * D, KD)], sems[b]
                    ).wait()

                def pos16(ii):
                    xvec = xv[pl.ds(pos0 + ii * _L, _L)]
                    svec = jnp.where(xvec > 0.0, ones, zeros)
                    for j in range(_L):
                        sj = jnp.broadcast_to(svec[j], (_L,))
                        o = (ii * _L + j) * D
                        for k in range(n_sub):
                            rows_b[pl.ds(o + k * _L, _L)] = t0[k] + sj * td[k]

                plsc.parallel_loop(0, K // _L, 1, unroll=4)(pos16)
                pltpu.async_copy(
                    rows_b, out_hbm.at[pl.ds((base + pos0) * D, KD)], sems[b]
                )
            return carry

        lax.fori_loop(0, n_outer, outer, 0)
        for b in range(NB):
            pltpu.make_async_copy(
                rows.at[b], out_hbm.at[pl.ds(base * D, KD)], sems[b]
            ).wait()

    return body(x_flat, table_flat)


def kernel(x, embedding):
    B, N = x.shape
    V, D = embedding.shape
    out = _sc_embed(x.reshape(B * N), embedding.reshape(V * D), D)
    return out.reshape(B, N, D)


# speedup vs baseline: 3.0203x; 1.7182x over previous
import functools

import jax
import jax.numpy as jnp
from jax import lax
from jax.experimental import pallas as pl
from jax.experimental.pallas import tpu as pltpu
from jax.experimental.pallas import tpu_sc as plsc

_L = 16


def _sc_embed(x_flat, table, D):
    (P,) = x_flat.shape
    info = plsc.get_sparse_core_info()
    NC, NS = info.num_cores, info.num_subcores
    NW = NC * NS
    per_w = P // NW
    K = 256
    KC = 192
    KG = K - KC
    NB = 2
    n_outer = per_w // (K * NB)
    n_sub = D // _L
    nqg = KG // _L

    mesh = plsc.VectorSubcoreMesh(core_axis_name="c", subcore_axis_name="s")

    @functools.partial(
        pl.kernel,
        mesh=mesh,
        out_type=jax.ShapeDtypeStruct((P, D), jnp.float32),
        scratch_types=[
            pltpu.VMEM((per_w,), jnp.float32),
            pltpu.VMEM((2, D), jnp.float32),
            pltpu.VMEM((NB, KC, D), jnp.float32),
            pltpu.VMEM((NB, KG, D), jnp.float32),
            pltpu.VMEM((NB, KG), jnp.int32),
            pltpu.VMEM_SHARED((2, D), jnp.float32),
            pltpu.SemaphoreType.DMA,
            pltpu.SemaphoreType.DMA,
            pltpu.SemaphoreType.DMA,
            pltpu.SemaphoreType.DMA,
        ],
    )
    def body(x_hbm, tbl_hbm, out_hbm, xv, tblv, rowsc, rowsg, idxg,
             stbl, semo0, semo1, semg0, semg1):
        wid = lax.axis_index("s") * NC + lax.axis_index("c")
        base = wid * per_w
        pltpu.sync_copy(x_hbm.at[pl.ds(base, per_w)], xv)
        pltpu.sync_copy(tbl_hbm.at[pl.ds(0, 2)], tblv)
        @pl.when(lax.axis_index("s") == 0)
        def _stage():
            pltpu.sync_copy(tbl_hbm.at[pl.ds(0, 2)], stbl)

        plsc.subcore_barrier()
        semo = (semo0, semo1)
        semg = (semg0, semg1)

        t0 = [tblv[0, pl.ds(k * _L, _L)] for k in range(n_sub)]
        td = [tblv[1, pl.ds(k * _L, _L)] - t0[k] for k in range(n_sub)]
        ones = jnp.full((_L,), 1.0, jnp.float32)
        zeros = jnp.full((_L,), 0.0, jnp.float32)
        onei = jnp.full((_L,), 1, jnp.int32)
        zeroi = jnp.full((_L,), 0, jnp.int32)

        def outer(c, carry):
            for b in range(NB):
                rows_b = rowsc.at[b]
                pos0 = c * (K * NB) + b * K

                @pl.when(c > 0)
                def _wait():
                    pltpu.make_async_copy(
                        rowsc.at[b], out_hbm.at[pl.ds(0, KC)], semo[b]
                    ).wait()
                    pltpu.make_async_copy(
                        rowsg.at[b], out_hbm.at[pl.ds(0, KG)], semo[b]
                    ).wait()

                for q in range(nqg):
                    xq = xv[pl.ds(pos0 + KC + q * _L, _L)]
                    idxg.at[b][pl.ds(q * _L, _L)] = jnp.where(
                        xq > 0.0, onei, zeroi
                    )
                pltpu.async_copy(stbl.at[idxg.at[b]], rowsg.at[b], semg[b])

                def pos16(ii):
                    xvec = xv[pl.ds(pos0 + ii * _L, _L)]
                    svec = jnp.where(xvec > 0.0, ones, zeros)
                    for j in range(_L):
                        sj = jnp.broadcast_to(svec[j], (_L,))
                        o = ii * _L + j
                        for k in range(n_sub):
                            rows_b[o, pl.ds(k * _L, _L)] = t0[k] + sj * td[k]

                plsc.parallel_loop(0, KC // _L, 1, unroll=4)(pos16)

                pltpu.make_async_copy(
                    stbl.at[idxg.at[b]], rowsg.at[b], semg[b]
                ).wait()
                pltpu.async_copy(
                    rows_b, out_hbm.at[pl.ds(base + pos0, KC)], semo[b]
                )
                pltpu.async_copy(
                    rowsg.at[b],
                    out_hbm.at[pl.ds(base + pos0 + KC, KG)],
                    semo[b],
                )
            return carry

        lax.fori_loop(0, n_outer, outer, 0)
        for b in range(NB):
            pltpu.make_async_copy(
                rowsc.at[b], out_hbm.at[pl.ds(0, KC)], semo[b]
            ).wait()
            pltpu.make_async_copy(
                rowsg.at[b], out_hbm.at[pl.ds(0, KG)], semo[b]
            ).wait()

    return body(x_flat, table)


def kernel(x, embedding):
    B, N = x.shape
    V, D = embedding.shape
    out = _sc_embed(x.reshape(B * N), embedding, D)
    return out.reshape(B, N, D)
